# baseline (speedup 1.0000x reference)
import functools
import math

import jax
import jax.numpy as jnp
from jax import lax
from jax.experimental import pallas as pl
from jax.experimental.pallas import tpu as pltpu
from jax.experimental.pallas import tpu_sc as plsc

D_MODEL = 64
LANES = 16
NUM_WORKERS = 32
XCHUNK = 8
IDX_PAD = 128
K_GATHER = 24
SCALE = math.sqrt(D_MODEL)


def _make_sc_lookup(n_x, k_x, d_model):
    assert d_model == D_MODEL
    assert k_x <= K_GATHER
    assert n_x % (NUM_WORKERS * 2 * XCHUNK) == 0
    xrows_per_w = n_x // NUM_WORKERS
    n_chunks = xrows_per_w // XCHUNK
    buf_rows = XCHUNK * K_GATHER

    mesh = plsc.VectorSubcoreMesh(core_axis_name="c", subcore_axis_name="s")

    @functools.partial(
        pl.kernel,
        mesh=mesh,
        out_type=jax.ShapeDtypeStruct((n_x * k_x, d_model), jnp.float32),
        compiler_params=pltpu.CompilerParams(use_tc_tiling_on_sc=False),
        scratch_types=[
            pltpu.VMEM((xrows_per_w, IDX_PAD), jnp.int32),
            pltpu.VMEM((buf_rows, d_model), jnp.float32),
            pltpu.VMEM((buf_rows, d_model), jnp.float32),
            pltpu.VMEM((XCHUNK * k_x, d_model), jnp.float32),
            pltpu.VMEM((XCHUNK * k_x, d_model), jnp.float32),
            pltpu.SemaphoreType.DMA,
            pltpu.SemaphoreType.DMA,
            pltpu.SemaphoreType.DMA,
            pltpu.SemaphoreType.DMA,
        ],
    )
    def sc_lookup(x_hbm, table_hbm, out_hbm, idx_v, rows0, rows1,
                  packed0, packed1, sem0, sem1, osem0, osem1):
        wid = lax.axis_index("s") * 2 + lax.axis_index("c")
        xrow_base = wid * xrows_per_w
        out_rows = XCHUNK * k_x
        out_base = wid * xrows_per_w * k_x

        rows = (rows0, rows1)
        packed = (packed0, packed1)
        sems = (sem0, sem1)
        osems = (osem0, osem1)

        pltpu.sync_copy(x_hbm.at[pl.ds(xrow_base, xrows_per_w)], idx_v)

        def fire(chunk, buf):
            for i in range(XCHUNK):
                pltpu.async_copy(
                    table_hbm.at[idx_v.at[chunk * XCHUNK + i, pl.ds(0, K_GATHER)]],
                    rows[buf].at[pl.ds(i * K_GATHER, K_GATHER)],
                    sems[buf],
                )

        def drain(buf):
            pltpu.make_async_copy(
                table_hbm.at[pl.ds(0, buf_rows)], rows[buf], sems[buf]
            ).wait()

        fire(0, 0)
        fire(1, 1)

        def drain_out(buf):
            pltpu.make_async_copy(
                table_hbm.at[pl.ds(0, out_rows)], packed[buf], osems[buf]
            ).wait()

        def chunk_body(i, carry):
            for buf in range(2):
                c = 2 * i + buf
                drain(buf)

                @pl.when(c >= 2)
                def _():
                    drain_out(buf)

                def scale_body(g, acc):
                    for q in range(4):
                        p = 4 * g + q
                        src = (K_GATHER - k_x) * (p // k_x) + p
                        for s in range(d_model // LANES):
                            lanes = pl.ds(s * LANES, LANES)
                            packed[buf][p, lanes] = rows[buf][src, lanes] * SCALE
                    return acc

                lax.fori_loop(0, out_rows // 4, scale_body, 0)

                pltpu.async_copy(
                    packed[buf],
                    out_hbm.at[pl.ds(out_base + c * out_rows, out_rows)],
                    osems[buf],
                )

                @pl.when(c + 2 < n_chunks)
                def _():
                    fire(c + 2, buf)
            return carry

        lax.fori_loop(0, n_chunks // 2, chunk_body, 0)
        drain_out(0)
        drain_out(1)

    return sc_lookup


def kernel(x, table):
    n_x, k_x = x.shape
    xp = jnp.pad(x.astype(jnp.int32), ((0, 0), (0, IDX_PAD - k_x)))
    out = _make_sc_lookup(n_x, k_x, table.shape[1])(xp, table)
    return out.reshape(n_x, k_x, D_MODEL)


# speedup vs baseline: 1.0027x; 1.0027x over previous
import functools
import math

import jax
import jax.numpy as jnp
from jax import lax
from jax.experimental import pallas as pl
from jax.experimental.pallas import tpu as pltpu
from jax.experimental.pallas import tpu_sc as plsc

D_MODEL = 64
LANES = 16
NUM_WORKERS = 32
XCHUNK = 8
IDX_PAD = 128
K_GATHER = 24
SCALE = math.sqrt(D_MODEL)


def _make_sc_lookup(n_x, k_x, d_model):
    assert d_model == D_MODEL
    assert k_x <= K_GATHER
    assert n_x % (NUM_WORKERS * 2 * XCHUNK) == 0
    xrows_per_w = n_x // NUM_WORKERS
    n_chunks = xrows_per_w // XCHUNK
    buf_rows = XCHUNK * K_GATHER

    mesh = plsc.VectorSubcoreMesh(core_axis_name="c", subcore_axis_name="s")

    @functools.partial(
        pl.kernel,
        mesh=mesh,
        out_type=jax.ShapeDtypeStruct((n_x * k_x, d_model), jnp.float32),
        compiler_params=pltpu.CompilerParams(use_tc_tiling_on_sc=False),
        scratch_types=[
            pltpu.VMEM((xrows_per_w, K_GATHER), jnp.int32),
            pltpu.VMEM((buf_rows, d_model), jnp.float32),
            pltpu.VMEM((buf_rows, d_model), jnp.float32),
            pltpu.VMEM((XCHUNK * k_x, d_model), jnp.float32),
            pltpu.VMEM((XCHUNK * k_x, d_model), jnp.float32),
            pltpu.SemaphoreType.DMA,
            pltpu.SemaphoreType.DMA,
            pltpu.SemaphoreType.DMA,
            pltpu.SemaphoreType.DMA,
        ],
    )
    def sc_lookup(x_hbm, table_hbm, out_hbm, idx_v, rows0, rows1,
                  packed0, packed1, sem0, sem1, osem0, osem1):
        wid = lax.axis_index("s") * 2 + lax.axis_index("c")
        xrow_base = wid * xrows_per_w
        out_rows = XCHUNK * k_x
        out_base = wid * xrows_per_w * k_x

        rows = (rows0, rows1)
        packed = (packed0, packed1)
        sems = (sem0, sem1)
        osems = (osem0, osem1)

        pltpu.sync_copy(
            x_hbm.at[pl.ds(xrow_base, xrows_per_w), pl.ds(0, K_GATHER)], idx_v
        )

        def fire(chunk, buf):
            for i in range(XCHUNK):
                pltpu.async_copy(
                    table_hbm.at[idx_v.at[chunk * XCHUNK + i]],
                    rows[buf].at[pl.ds(i * K_GATHER, K_GATHER)],
                    sems[buf],
                )

        def drain(buf):
            pltpu.make_async_copy(
                table_hbm.at[pl.ds(0, buf_rows)], rows[buf], sems[buf]
            ).wait()

        fire(0, 0)
        fire(1, 1)

        def drain_out(buf):
            pltpu.make_async_copy(
                table_hbm.at[pl.ds(0, out_rows)], packed[buf], osems[buf]
            ).wait()

        def chunk_body(i, carry):
            for buf in range(2):
                c = 2 * i + buf
                drain(buf)

                @pl.when(c >= 2)
                def _():
                    drain_out(buf)

                def scale_body(g, acc):
                    for q in range(4):
                        p = 4 * g + q
                        src = (K_GATHER - k_x) * (p // k_x) + p
                        for s in range(d_model // LANES):
                            lanes = pl.ds(s * LANES, LANES)
                            packed[buf][p, lanes] = rows[buf][src, lanes] * SCALE
                    return acc

                lax.fori_loop(0, out_rows // 4, scale_body, 0)

                pltpu.async_copy(
                    packed[buf],
                    out_hbm.at[pl.ds(out_base + c * out_rows, out_rows)],
                    osems[buf],
                )

                @pl.when(c + 2 < n_chunks)
                def _():
                    fire(c + 2, buf)
            return carry

        lax.fori_loop(0, n_chunks // 2, chunk_body, 0)
        drain_out(0)
        drain_out(1)

    return sc_lookup


def kernel(x, table):
    n_x, k_x = x.shape
    xp = jnp.pad(x.astype(jnp.int32), ((0, 0), (0, IDX_PAD - k_x)))
    out = _make_sc_lookup(n_x, k_x, table.shape[1])(xp, table)
    return out.reshape(n_x, k_x, D_MODEL)


# speedup vs baseline: 2.5491x; 2.5421x over previous
import functools
import math

import jax
import jax.numpy as jnp
from jax import lax
from jax.experimental import pallas as pl
from jax.experimental.pallas import tpu as pltpu
from jax.experimental.pallas import tpu_sc as plsc

D_MODEL = 64
LANES = 16
NUM_WORKERS = 32
IDX_MINOR = 128
GATHERS_PER_CHUNK = 4
CHUNK = IDX_MINOR * GATHERS_PER_CHUNK
SCALE = math.sqrt(D_MODEL)


def _make_sc_lookup(batch, d_model):
    assert d_model == D_MODEL
    assert batch % (NUM_WORKERS * CHUNK) == 0
    rows_per_w = batch // NUM_WORKERS
    idx_rows_per_w = rows_per_w // IDX_MINOR
    n_chunks = rows_per_w // CHUNK

    mesh = plsc.VectorSubcoreMesh(core_axis_name="c", subcore_axis_name="s")

    @functools.partial(
        pl.kernel,
        mesh=mesh,
        out_type=jax.ShapeDtypeStruct((batch, d_model), jnp.float32),
        compiler_params=pltpu.CompilerParams(use_tc_tiling_on_sc=False),
        scratch_types=[
            pltpu.VMEM((idx_rows_per_w, IDX_MINOR), jnp.int32),
            pltpu.VMEM((CHUNK, D_MODEL), jnp.float32),
            pltpu.VMEM((CHUNK, D_MODEL), jnp.float32),
            pltpu.SemaphoreType.DMA,
            pltpu.SemaphoreType.DMA,
        ],
    )
    def sc_lookup(idx_hbm, table_hbm, out_hbm, idx_v, rows0, rows1, sem0, sem1):
        wid = lax.axis_index("s") * 2 + lax.axis_index("c")
        idx_row_base = wid * idx_rows_per_w
        out_base = wid * rows_per_w

        rows = (rows0, rows1)
        sems = (sem0, sem1)

        pltpu.sync_copy(idx_hbm.at[pl.ds(idx_row_base, idx_rows_per_w)], idx_v)

        def fire(chunk, buf):
            for g in range(GATHERS_PER_CHUNK):
                pltpu.async_copy(
                    table_hbm.at[idx_v.at[chunk * GATHERS_PER_CHUNK + g]],
                    rows[buf].at[pl.ds(g * IDX_MINOR, IDX_MINOR)],
                    sems[buf],
                )

        def drain(chunk, buf):
            for g in range(GATHERS_PER_CHUNK):
                pltpu.make_async_copy(
                    table_hbm.at[idx_v.at[chunk * GATHERS_PER_CHUNK + g]],
                    rows[buf].at[pl.ds(g * IDX_MINOR, IDX_MINOR)],
                    sems[buf],
                ).wait()

        fire(0, 0)
        fire(1, 1)

        def chunk_body(i, carry):
            for buf in range(2):
                c = 2 * i + buf
                drain(c, buf)

                def scale_body(r, acc):
                    for rr in range(4):
                        for s in range(D_MODEL // LANES):
                            sl = (4 * r + rr, pl.ds(s * LANES, LANES))
                            rows[buf][sl] = rows[buf][sl] * SCALE
                    return acc

                lax.fori_loop(0, CHUNK // 4, scale_body, 0)

                pltpu.sync_copy(
                    rows[buf],
                    out_hbm.at[pl.ds(out_base + c * CHUNK, CHUNK)],
                )

                @pl.when(c + 2 < n_chunks)
                def _():
                    fire(c + 2, buf)
            return carry

        lax.fori_loop(0, n_chunks // 2, chunk_body, 0)

    return sc_lookup


def kernel(x, table):
    batch = x.shape[0] * x.shape[1]
    xi = x.reshape(batch).astype(jnp.int32).reshape(batch // IDX_MINOR, IDX_MINOR)
    out = _make_sc_lookup(batch, table.shape[1])(xi, table)
    return out.reshape(x.shape[0], x.shape[1], D_MODEL)
